# Initial kernel scaffold; baseline (speedup 1.0000x reference)
import jax
import jax.numpy as jnp
from jax.experimental import pallas as pl


def kernel(x, W1, b1, W2, b2, bn_w, bn_b, keys, values):
    raise NotImplementedError("write your pallas kernel here")


# speedup vs baseline: 5.5974x; 5.5974x over previous
import functools
import math

import jax
import jax.numpy as jnp
from jax.experimental import pallas as pl

_HID = 1024
_KD = 512
_HALF = 256
_HEADS = 4
_SUB = 256
_KNN = 16

_INTERPRET = False


def _mlp_kernel(x_ref, w1t_ref, b1_ref, w2t_ref, b2_ref, q_ref, s_ref, ss_ref):
    h = jnp.maximum(
        jnp.dot(x_ref[...], w1t_ref[...], preferred_element_type=jnp.float32)
        + b1_ref[...], 0.0)
    q = jnp.dot(h, w2t_ref[...], preferred_element_type=jnp.float32) + b2_ref[...]
    q_ref[...] = q
    s_ref[0] = jnp.sum(q, axis=0, keepdims=True)
    ss_ref[0] = jnp.sum(q * q, axis=0, keepdims=True)


def _run_mlp(x2, W1, b1, W2, b2, tb):
    n = x2.shape[0]
    grid = n // tb
    return pl.pallas_call(
        _mlp_kernel,
        grid=(grid,),
        in_specs=[
            pl.BlockSpec((tb, _HID), lambda i: (i, 0)),
            pl.BlockSpec((_HID, _KD), lambda i: (0, 0)),
            pl.BlockSpec((1, _KD), lambda i: (0, 0)),
            pl.BlockSpec((_KD, 2 * _HID), lambda i: (0, 0)),
            pl.BlockSpec((1, 2 * _HID), lambda i: (0, 0)),
        ],
        out_specs=[
            pl.BlockSpec((tb, 2 * _HID), lambda i: (i, 0)),
            pl.BlockSpec((1, 1, 2 * _HID), lambda i: (i, 0, 0)),
            pl.BlockSpec((1, 1, 2 * _HID), lambda i: (i, 0, 0)),
        ],
        out_shape=[
            jax.ShapeDtypeStruct((n, 2 * _HID), jnp.float32),
            jax.ShapeDtypeStruct((grid, 1, 2 * _HID), jnp.float32),
            jax.ShapeDtypeStruct((grid, 1, 2 * _HID), jnp.float32),
        ],
        interpret=_INTERPRET,
    )(x2, W1.T, b1.reshape(1, _KD), W2.T, b2.reshape(1, 2 * _HID))


def _topk16(s):
    lane = jax.lax.broadcasted_iota(jnp.int32, s.shape, 1)
    vals, idxs = [], []
    cur = s
    for _ in range(_KNN):
        m = jnp.max(cur, axis=1, keepdims=True)
        idx = jnp.min(jnp.where(cur == m, lane, s.shape[1]), axis=1, keepdims=True)
        vals.append(m)
        idxs.append(idx)
        cur = jnp.where(lane == idx, -jnp.inf, cur)
    return (jnp.concatenate(vals, axis=1),
            jnp.concatenate(idxs, axis=1).astype(jnp.float32))


def _topk16_payload(s, pa, pb):
    lane = jax.lax.broadcasted_iota(jnp.int32, s.shape, 1)
    vals, pas, pbs = [], [], []
    cur = s
    for _ in range(_KNN):
        m = jnp.max(cur, axis=1, keepdims=True)
        idx = jnp.min(jnp.where(cur == m, lane, s.shape[1]), axis=1, keepdims=True)
        hit = lane == idx
        vals.append(m)
        pas.append(jnp.sum(jnp.where(hit, pa, 0.0), axis=1, keepdims=True))
        pbs.append(jnp.sum(jnp.where(hit, pb, 0.0), axis=1, keepdims=True))
        cur = jnp.where(hit, -jnp.inf, cur)
    return (jnp.concatenate(vals, axis=1),
            jnp.concatenate(pas, axis=1),
            jnp.concatenate(pbs, axis=1))


def _score_kernel(q_ref, sc_ref, sh_ref, k1t_ref, k2t_ref,
                  w_ref, a_ref, b_ref):
    qn = q_ref[...] * sc_ref[...] + sh_ref[...]
    s1 = jnp.dot(qn[:, :_HALF], k1t_ref[0], preferred_element_type=jnp.float32)
    s2 = jnp.dot(qn[:, _HALF:], k2t_ref[0], preferred_element_type=jnp.float32)
    v1, i1 = _topk16(s1)
    v2, i2 = _topk16(s2)
    lane = jax.lax.broadcasted_iota(jnp.int32, (_KNN, _KNN * _KNN), 1)
    row = jax.lax.broadcasted_iota(jnp.int32, (_KNN, _KNN * _KNN), 0)
    selA = (lane // _KNN == row).astype(jnp.float32)
    selB = (lane % _KNN == row).astype(jnp.float32)
    all_s = (jnp.dot(v1, selA, preferred_element_type=jnp.float32, precision=jax.lax.Precision.HIGHEST)
             + jnp.dot(v2, selB, preferred_element_type=jnp.float32, precision=jax.lax.Precision.HIGHEST))
    ai = jnp.dot(i1, selA, preferred_element_type=jnp.float32, precision=jax.lax.Precision.HIGHEST)
    bi = jnp.dot(i2, selB, preferred_element_type=jnp.float32, precision=jax.lax.Precision.HIGHEST)
    vt, at, bt = _topk16_payload(all_s, ai, bi)
    m = jnp.max(vt, axis=1, keepdims=True)
    e = jnp.exp(vt - m)
    w = e / jnp.sum(e, axis=1, keepdims=True)
    w_ref[0] = w
    a_ref[0] = at
    b_ref[0] = bt


def _run_score(q_bykey, scale_full, shift_full, k1t, k2t, tb):
    n = q_bykey.shape[0]
    per_key = n // _HEADS
    grid_g = per_key // tb
    ng = _HEADS * _HEADS
    return pl.pallas_call(
        _score_kernel,
        grid=(_HEADS, _HEADS, grid_g),
        in_specs=[
            pl.BlockSpec((tb, _KD), lambda k, h, g: (k * (per_key // tb) + g, h)),
            pl.BlockSpec((1, _KD), lambda k, h, g: (0, h)),
            pl.BlockSpec((1, _KD), lambda k, h, g: (0, h)),
            pl.BlockSpec((1, _HALF, _SUB), lambda k, h, g: (k, 0, 0)),
            pl.BlockSpec((1, _HALF, _SUB), lambda k, h, g: (k, 0, 0)),
        ],
        out_specs=[
            pl.BlockSpec((1, tb, _KNN), lambda k, h, g: (k * _HEADS + h, g, 0)),
            pl.BlockSpec((1, tb, _KNN), lambda k, h, g: (k * _HEADS + h, g, 0)),
            pl.BlockSpec((1, tb, _KNN), lambda k, h, g: (k * _HEADS + h, g, 0)),
        ],
        out_shape=[
            jax.ShapeDtypeStruct((ng, per_key, _KNN), jnp.float32),
            jax.ShapeDtypeStruct((ng, per_key, _KNN), jnp.float32),
            jax.ShapeDtypeStruct((ng, per_key, _KNN), jnp.float32),
        ],
        interpret=_INTERPRET,
    )(q_bykey, scale_full, shift_full, k1t, k2t)


def _scatter_kernel(w_ref, a_ref, b_ref, acc_ref):
    @pl.when(jnp.logical_and(pl.program_id(0) == 0, pl.program_id(1) == 0))
    def _init():
        acc_ref[...] = jnp.zeros_like(acc_ref)

    w = w_ref[0]
    a = a_ref[0].astype(jnp.int32)
    b = b_ref[0].astype(jnp.int32)
    rows = w.shape[0]
    n_iota = jax.lax.broadcasted_iota(jnp.int32, (rows, _SUB), 1)
    acc = jnp.zeros((_SUB, _SUB), jnp.float32)
    for c in range(_KNN):
        oa = jnp.where(n_iota == a[:, c:c + 1], w[:, c:c + 1], 0.0)
        ob = (n_iota == b[:, c:c + 1]).astype(jnp.float32)
        acc = acc + jax.lax.dot_general(
            oa, ob, (((0,), (0,)), ((), ())),
            preferred_element_type=jnp.float32,
            precision=jax.lax.Precision.HIGHEST)
    acc_ref[...] += acc


def _run_scatter(wts, aidx, bidx, tb):
    n = wts.shape[1]
    grid_t = n // tb
    return pl.pallas_call(
        _scatter_kernel,
        grid=(wts.shape[0], grid_t),
        in_specs=[
            pl.BlockSpec((1, tb, _KNN), lambda h, t: (h, t, 0)),
            pl.BlockSpec((1, tb, _KNN), lambda h, t: (h, t, 0)),
            pl.BlockSpec((1, tb, _KNN), lambda h, t: (h, t, 0)),
        ],
        out_specs=pl.BlockSpec((_SUB, _SUB), lambda h, t: (0, 0)),
        out_shape=jax.ShapeDtypeStruct((_SUB, _SUB), jnp.float32),
        interpret=_INTERPRET,
    )(wts, aidx, bidx)


def _matvec_kernel(w_ref, v_ref, o_ref):
    @pl.when(pl.program_id(0) == 0)
    def _init():
        o_ref[...] = jnp.zeros_like(o_ref)

    o_ref[...] += jnp.dot(w_ref[...], v_ref[...],
                          preferred_element_type=jnp.float32, precision=jax.lax.Precision.HIGHEST)


def _run_matvec(w_flat, values, rb):
    nrows = values.shape[0]
    grid = nrows // rb
    return pl.pallas_call(
        _matvec_kernel,
        grid=(grid,),
        in_specs=[
            pl.BlockSpec((1, rb), lambda i: (0, i)),
            pl.BlockSpec((rb, _HID), lambda i: (i, 0)),
        ],
        out_specs=pl.BlockSpec((1, _HID), lambda i: (0, 0)),
        out_shape=jax.ShapeDtypeStruct((1, _HID), jnp.float32),
        interpret=_INTERPRET,
    )(w_flat, values)


def kernel(x, W1, b1, W2, b2, bn_w, bn_b, keys, values):
    Bx, Sx, H = x.shape
    x2 = x.reshape(Bx * Sx, H)
    n = x2.shape[0]

    q, psum, psumsq = _run_mlp(x2, W1, b1, W2, b2, tb=256)

    tot = n * _HEADS
    csum = psum.reshape(-1, 2 * _HID).sum(axis=0).reshape(_HEADS, _KD).sum(axis=0)
    csumsq = psumsq.reshape(-1, 2 * _HID).sum(axis=0).reshape(_HEADS, _KD).sum(axis=0)
    mean = csum / tot
    var = csumsq / tot - mean * mean
    scale = bn_w / jnp.sqrt(var + 1e-5)
    shift = bn_b - mean * scale
    scale_full = jnp.tile(scale, _HEADS).reshape(1, 2 * _HID)
    shift_full = jnp.tile(shift, _HEADS).reshape(1, 2 * _HID)

    k1t = jnp.transpose(keys[:, 0], (0, 2, 1))
    k2t = jnp.transpose(keys[:, 1], (0, 2, 1))

    q_bykey = q.reshape(n // _HEADS, _HEADS, 2 * _HID)
    q_bykey = jnp.transpose(q_bykey, (1, 0, 2)).reshape(n, 2 * _HID)

    tb2 = min(256, n // _HEADS)
    wts, aidx, bidx = _run_score(q_bykey, scale_full, shift_full, k1t, k2t,
                                 tb=tb2)

    wacc = _run_scatter(wts, aidx, bidx, tb=tb2)

    out = _run_matvec(wacc.reshape(1, _SUB * _SUB), values, rb=2048)
    return out.reshape(Bx, _HID)


# speedup vs baseline: 7.4777x; 1.3359x over previous
import functools

import jax
import jax.numpy as jnp
from jax.experimental import pallas as pl
from jax.experimental.pallas import tpu as pltpu
from jax.experimental.pallas import tpu_sc as plsc

_HID = 1024
_KD = 512
_HALF = 256
_HEADS = 4
_SUB = 256
_KNN = 16
_NROWS = _SUB * _SUB

_SC_CORES = 2
_SC_SUBCORES = 16
_SC_WORKERS = _SC_CORES * _SC_SUBCORES

_INTERPRET = False


def _mlp_kernel(x_ref, w1t_ref, b1_ref, w2t_ref, b2_ref, q_ref, s_ref, ss_ref):
    h = jnp.maximum(
        jnp.dot(x_ref[...], w1t_ref[...], preferred_element_type=jnp.float32)
        + b1_ref[...], 0.0)
    q = jnp.dot(h, w2t_ref[...], preferred_element_type=jnp.float32) + b2_ref[...]
    q_ref[...] = q
    s_ref[0] = jnp.sum(q, axis=0, keepdims=True)
    ss_ref[0] = jnp.sum(q * q, axis=0, keepdims=True)


def _run_mlp(x2, W1, b1, W2, b2, tb):
    n = x2.shape[0]
    grid = n // tb
    return pl.pallas_call(
        _mlp_kernel,
        grid=(grid,),
        in_specs=[
            pl.BlockSpec((tb, _HID), lambda i: (i, 0)),
            pl.BlockSpec((_HID, _KD), lambda i: (0, 0)),
            pl.BlockSpec((1, _KD), lambda i: (0, 0)),
            pl.BlockSpec((_KD, 2 * _HID), lambda i: (0, 0)),
            pl.BlockSpec((1, 2 * _HID), lambda i: (0, 0)),
        ],
        out_specs=[
            pl.BlockSpec((tb, 2 * _HID), lambda i: (i, 0)),
            pl.BlockSpec((1, 1, 2 * _HID), lambda i: (i, 0, 0)),
            pl.BlockSpec((1, 1, 2 * _HID), lambda i: (i, 0, 0)),
        ],
        out_shape=[
            jax.ShapeDtypeStruct((n, 2 * _HID), jnp.float32),
            jax.ShapeDtypeStruct((grid, 1, 2 * _HID), jnp.float32),
            jax.ShapeDtypeStruct((grid, 1, 2 * _HID), jnp.float32),
        ],
        interpret=_INTERPRET,
    )(x2, W1.T, b1.reshape(1, _KD), W2.T, b2.reshape(1, 2 * _HID))


def _topk16(s):
    lane = jax.lax.broadcasted_iota(jnp.int32, s.shape, 1)
    vals, idxs = [], []
    cur = s
    for _ in range(_KNN):
        m = jnp.max(cur, axis=1, keepdims=True)
        idx = jnp.min(jnp.where(cur == m, lane, s.shape[1]), axis=1, keepdims=True)
        vals.append(m)
        idxs.append(idx)
        cur = jnp.where(lane == idx, -jnp.inf, cur)
    return (jnp.concatenate(vals, axis=1),
            jnp.concatenate(idxs, axis=1).astype(jnp.float32))


def _topk16_payload(s, pa):
    lane = jax.lax.broadcasted_iota(jnp.int32, s.shape, 1)
    vals, pas = [], []
    cur = s
    for _ in range(_KNN):
        m = jnp.max(cur, axis=1, keepdims=True)
        idx = jnp.min(jnp.where(cur == m, lane, s.shape[1]), axis=1, keepdims=True)
        hit = lane == idx
        vals.append(m)
        pas.append(jnp.sum(jnp.where(hit, pa, 0.0), axis=1, keepdims=True))
        cur = jnp.where(hit, -jnp.inf, cur)
    return (jnp.concatenate(vals, axis=1), jnp.concatenate(pas, axis=1))


def _score_kernel(q_ref, sc_ref, sh_ref, k1t_ref, k2t_ref, w_ref, ci_ref):
    qn = q_ref[...] * sc_ref[...] + sh_ref[...]
    s1 = jnp.dot(qn[:, :_HALF], k1t_ref[0], preferred_element_type=jnp.float32)
    s2 = jnp.dot(qn[:, _HALF:], k2t_ref[0], preferred_element_type=jnp.float32)
    v1, i1 = _topk16(s1)
    v2, i2 = _topk16(s2)
    lane = jax.lax.broadcasted_iota(jnp.int32, (_KNN, _KNN * _KNN), 1)
    row = jax.lax.broadcasted_iota(jnp.int32, (_KNN, _KNN * _KNN), 0)
    selA = (lane // _KNN == row).astype(jnp.float32)
    selB = (lane % _KNN == row).astype(jnp.float32)
    hp = jax.lax.Precision.HIGHEST
    all_s = (jnp.dot(v1, selA, preferred_element_type=jnp.float32, precision=hp)
             + jnp.dot(v2, selB, preferred_element_type=jnp.float32, precision=hp))
    cidx = (jnp.dot(i1, selA, preferred_element_type=jnp.float32, precision=hp)
            * _SUB
            + jnp.dot(i2, selB, preferred_element_type=jnp.float32, precision=hp))
    vt, ct = _topk16_payload(all_s, cidx)
    m = jnp.max(vt, axis=1, keepdims=True)
    e = jnp.exp(vt - m)
    w_ref[0] = e / jnp.sum(e, axis=1, keepdims=True)
    ci_ref[0] = ct.astype(jnp.int32)


def _run_score(q_bykey, scale_full, shift_full, k1t, k2t, tb):
    n = q_bykey.shape[0]
    per_key = n // _HEADS
    grid_g = per_key // tb
    ng = _HEADS * _HEADS
    return pl.pallas_call(
        _score_kernel,
        grid=(_HEADS, _HEADS, grid_g),
        in_specs=[
            pl.BlockSpec((tb, _KD), lambda k, h, g: (k * (per_key // tb) + g, h)),
            pl.BlockSpec((1, _KD), lambda k, h, g: (0, h)),
            pl.BlockSpec((1, _KD), lambda k, h, g: (0, h)),
            pl.BlockSpec((1, _HALF, _SUB), lambda k, h, g: (k, 0, 0)),
            pl.BlockSpec((1, _HALF, _SUB), lambda k, h, g: (k, 0, 0)),
        ],
        out_specs=[
            pl.BlockSpec((1, tb, _KNN), lambda k, h, g: (k * _HEADS + h, g, 0)),
            pl.BlockSpec((1, tb, _KNN), lambda k, h, g: (k * _HEADS + h, g, 0)),
        ],
        out_shape=[
            jax.ShapeDtypeStruct((ng, per_key, _KNN), jnp.float32),
            jax.ShapeDtypeStruct((ng, per_key, _KNN), jnp.int32),
        ],
        interpret=_INTERPRET,
    )(q_bykey, scale_full, shift_full, k1t, k2t)


def _make_sc_scatter(npairs):
    chunk = npairs // _SC_WORKERS
    mesh = plsc.VectorSubcoreMesh(core_axis_name="c", subcore_axis_name="s",
                                  num_cores=_SC_CORES,
                                  num_subcores=_SC_SUBCORES)

    @functools.partial(
        pl.kernel, mesh=mesh,
        compiler_params=pltpu.CompilerParams(needs_layout_passes=False),
        out_type=jax.ShapeDtypeStruct((_SC_WORKERS, _NROWS), jnp.float32),
        scratch_types=[
            pltpu.VMEM((chunk,), jnp.int32),
            pltpu.VMEM((chunk,), jnp.float32),
            pltpu.VMEM((_NROWS,), jnp.float32),
        ],
    )
    def sc_scatter(idx_hbm, w_hbm, zero_hbm, out_hbm, idx_v, w_v, acc_v):
        wid = (jax.lax.axis_index("s") * _SC_CORES + jax.lax.axis_index("c"))
        base = wid * chunk
        pltpu.sync_copy(idx_hbm.at[pl.ds(base, chunk)], idx_v)
        pltpu.sync_copy(w_hbm.at[pl.ds(base, chunk)], w_v)
        pltpu.sync_copy(zero_hbm, acc_v)

        def body(i, carry):
            off = pl.multiple_of(i * _KNN, 16)
            iv = idx_v[pl.ds(off, _KNN)]
            wv = w_v[pl.ds(off, _KNN)]
            plsc.addupdate_scatter(acc_v, [iv], wv)
            return carry

        jax.lax.fori_loop(0, chunk // _KNN, body, 0)
        pltpu.sync_copy(acc_v, out_hbm.at[wid])

    return sc_scatter


def _matvec_kernel(w_ref, v_ref, o_ref):
    @pl.when(pl.program_id(0) == 0)
    def _init():
        o_ref[...] = jnp.zeros_like(o_ref)

    wsum = jnp.sum(w_ref[...], axis=0)
    prod = v_ref[...] * wsum[:, :, None]
    o_ref[...] += jnp.sum(prod, axis=(0, 1)).reshape(1, _HID)


def _run_matvec(w32, values3, rb):
    nblk = values3.shape[0] // rb
    return pl.pallas_call(
        _matvec_kernel,
        grid=(nblk,),
        in_specs=[
            pl.BlockSpec((_SC_WORKERS, rb, 128), lambda i: (0, i, 0)),
            pl.BlockSpec((rb, 128, _HID), lambda i: (i, 0, 0)),
        ],
        out_specs=pl.BlockSpec((1, _HID), lambda i: (0, 0)),
        out_shape=jax.ShapeDtypeStruct((1, _HID), jnp.float32),
        interpret=_INTERPRET,
    )(w32, values3)


def kernel(x, W1, b1, W2, b2, bn_w, bn_b, keys, values):
    Bx, Sx, H = x.shape
    x2 = x.reshape(Bx * Sx, H)
    n = x2.shape[0]

    x_perm = jnp.transpose(x2.reshape(n // _HEADS, _HEADS, H), (1, 0, 2))
    x_perm = x_perm.reshape(n, H)

    q, psum, psumsq = _run_mlp(x_perm, W1, b1, W2, b2, tb=256)

    tot = n * _HEADS
    csum = psum.reshape(-1, 2 * _HID).sum(axis=0).reshape(_HEADS, _KD).sum(axis=0)
    csumsq = psumsq.reshape(-1, 2 * _HID).sum(axis=0).reshape(_HEADS, _KD).sum(axis=0)
    mean = csum / tot
    var = csumsq / tot - mean * mean
    scale = bn_w / jnp.sqrt(var + 1e-5)
    shift = bn_b - mean * scale
    scale_full = jnp.tile(scale, _HEADS).reshape(1, 2 * _HID)
    shift_full = jnp.tile(shift, _HEADS).reshape(1, 2 * _HID)

    k1t = jnp.transpose(keys[:, 0], (0, 2, 1))
    k2t = jnp.transpose(keys[:, 1], (0, 2, 1))

    tb2 = min(256, n // _HEADS)
    wts, cidx = _run_score(q, scale_full, shift_full, k1t, k2t, tb=tb2)

    npairs = wts.size
    zero = jnp.zeros((_NROWS,), jnp.float32)
    w32 = _make_sc_scatter(npairs)(cidx.reshape(-1), wts.reshape(-1), zero)

    out = _run_matvec(w32.reshape(_SC_WORKERS, 512, 128),
                      values.reshape(512, 128, _HID), rb=16)
    return out.reshape(Bx, _HID)


# speedup vs baseline: 8.4837x; 1.1345x over previous
import functools

import jax
import jax.numpy as jnp
from jax.experimental import pallas as pl
from jax.experimental.pallas import tpu as pltpu
from jax.experimental.pallas import tpu_sc as plsc

_HID = 1024
_KD = 512
_HALF = 256
_HEADS = 4
_SUB = 256
_KNN = 16
_NROWS = _SUB * _SUB

_SC_CORES = 2
_SC_SUBCORES = 16
_SC_WORKERS = _SC_CORES * _SC_SUBCORES

_INTERPRET = False


def _mlp_kernel(x_ref, w1t_ref, b1_ref, w2t_ref, b2_ref, q_ref, s_ref, ss_ref):
    h = jnp.maximum(
        jnp.dot(x_ref[...], w1t_ref[...], preferred_element_type=jnp.float32)
        + b1_ref[...], 0.0)
    q = jnp.dot(h, w2t_ref[...], preferred_element_type=jnp.float32) + b2_ref[...]
    q_ref[...] = q
    s_ref[0] = jnp.sum(q, axis=0, keepdims=True)
    ss_ref[0] = jnp.sum(q * q, axis=0, keepdims=True)


def _run_mlp(x2, W1, b1, W2, b2, tb):
    n = x2.shape[0]
    grid = n // tb
    return pl.pallas_call(
        _mlp_kernel,
        grid=(grid,),
        in_specs=[
            pl.BlockSpec((tb, _HID), lambda i: (i, 0)),
            pl.BlockSpec((_HID, _KD), lambda i: (0, 0)),
            pl.BlockSpec((1, _KD), lambda i: (0, 0)),
            pl.BlockSpec((_KD, 2 * _HID), lambda i: (0, 0)),
            pl.BlockSpec((1, 2 * _HID), lambda i: (0, 0)),
        ],
        out_specs=[
            pl.BlockSpec((tb, 2 * _HID), lambda i: (i, 0)),
            pl.BlockSpec((1, 1, 2 * _HID), lambda i: (i, 0, 0)),
            pl.BlockSpec((1, 1, 2 * _HID), lambda i: (i, 0, 0)),
        ],
        out_shape=[
            jax.ShapeDtypeStruct((n, 2 * _HID), jnp.float32),
            jax.ShapeDtypeStruct((grid, 1, 2 * _HID), jnp.float32),
            jax.ShapeDtypeStruct((grid, 1, 2 * _HID), jnp.float32),
        ],
        interpret=_INTERPRET,
    )(x2, W1.T, b1.reshape(1, _KD), W2.T, b2.reshape(1, 2 * _HID))


def _topk16(s):
    b = jax.lax.bitcast_convert_type(s, jnp.int32)
    k = b ^ (jax.lax.shift_right_arithmetic(b, 31) & jnp.int32(0x7FFFFFFF))
    lane = jax.lax.broadcasted_iota(jnp.int32, s.shape, 1)
    cur = (k & jnp.int32(~0xFF)) | (jnp.int32(255) - lane)
    ms = []
    for _ in range(_KNN):
        m = jnp.max(cur, axis=1, keepdims=True)
        ms.append(m)
        cur = jnp.where(cur == m, jnp.int32(-2147483647 - 1), cur)
    mk = jnp.concatenate(ms, axis=1)
    idx = jnp.int32(255) - (mk & jnp.int32(0xFF))
    vb = mk | jnp.int32(0xFF)
    vb = vb ^ (jax.lax.shift_right_arithmetic(vb, 31) & jnp.int32(0x7FFFFFFF))
    vals = jax.lax.bitcast_convert_type(vb, jnp.float32)
    return vals, idx


def _sel16(sel, payload):
    out = jnp.zeros_like(payload)
    for c in range(_KNN):
        out = out + jnp.where(sel == c, payload[:, c:c + 1], 0.0)
    return out


def _score_kernel(q_ref, sc_ref, sh_ref, k1t_ref, k2t_ref, w_ref, ci_ref):
    qn = q_ref[...] * sc_ref[...] + sh_ref[...]
    s1 = jnp.dot(qn[:, :_HALF], k1t_ref[0], preferred_element_type=jnp.float32)
    s2 = jnp.dot(qn[:, _HALF:], k2t_ref[0], preferred_element_type=jnp.float32)
    v1, i1 = _topk16(s1)
    v2, i2 = _topk16(s2)
    lane = jax.lax.broadcasted_iota(jnp.int32, (_KNN, _KNN * _KNN), 1)
    row = jax.lax.broadcasted_iota(jnp.int32, (_KNN, _KNN * _KNN), 0)
    selA = (lane // _KNN == row).astype(jnp.float32)
    selB = (lane % _KNN == row).astype(jnp.float32)
    hp = jax.lax.Precision.HIGHEST
    all_s = (jnp.dot(v1, selA, preferred_element_type=jnp.float32, precision=hp)
             + jnp.dot(v2, selB, preferred_element_type=jnp.float32, precision=hp))
    vt, lt = _topk16(all_s)
    at = _sel16(lt // _KNN, i1.astype(jnp.float32))
    bt = _sel16(lt % _KNN, i2.astype(jnp.float32))
    m = jnp.max(vt, axis=1, keepdims=True)
    e = jnp.exp(vt - m)
    w_ref[0] = e / jnp.sum(e, axis=1, keepdims=True)
    ci_ref[0] = (at * _SUB + bt).astype(jnp.int32)


def _run_score(q_bykey, scale_full, shift_full, k1t, k2t, tb):
    n = q_bykey.shape[0]
    per_key = n // _HEADS
    grid_g = per_key // tb
    ng = _HEADS * _HEADS
    return pl.pallas_call(
        _score_kernel,
        grid=(_HEADS, _HEADS, grid_g),
        in_specs=[
            pl.BlockSpec((tb, _KD), lambda k, h, g: (k * (per_key // tb) + g, h)),
            pl.BlockSpec((1, _KD), lambda k, h, g: (0, h)),
            pl.BlockSpec((1, _KD), lambda k, h, g: (0, h)),
            pl.BlockSpec((1, _HALF, _SUB), lambda k, h, g: (k, 0, 0)),
            pl.BlockSpec((1, _HALF, _SUB), lambda k, h, g: (k, 0, 0)),
        ],
        out_specs=[
            pl.BlockSpec((1, tb, _KNN), lambda k, h, g: (k * _HEADS + h, g, 0)),
            pl.BlockSpec((1, tb, _KNN), lambda k, h, g: (k * _HEADS + h, g, 0)),
        ],
        out_shape=[
            jax.ShapeDtypeStruct((ng, per_key, _KNN), jnp.float32),
            jax.ShapeDtypeStruct((ng, per_key, _KNN), jnp.int32),
        ],
        interpret=_INTERPRET,
    )(q_bykey, scale_full, shift_full, k1t, k2t)


def _make_sc_scatter(npairs):
    chunk = npairs // _SC_WORKERS
    mesh = plsc.VectorSubcoreMesh(core_axis_name="c", subcore_axis_name="s",
                                  num_cores=_SC_CORES,
                                  num_subcores=_SC_SUBCORES)

    @functools.partial(
        pl.kernel, mesh=mesh,
        compiler_params=pltpu.CompilerParams(needs_layout_passes=False),
        out_type=jax.ShapeDtypeStruct((_SC_WORKERS, _NROWS), jnp.float32),
        scratch_types=[
            pltpu.VMEM((chunk,), jnp.int32),
            pltpu.VMEM((chunk,), jnp.float32),
            pltpu.VMEM((_NROWS,), jnp.float32),
        ],
    )
    def sc_scatter(idx_hbm, w_hbm, zero_hbm, out_hbm, idx_v, w_v, acc_v):
        wid = (jax.lax.axis_index("s") * _SC_CORES + jax.lax.axis_index("c"))
        base = wid * chunk
        pltpu.sync_copy(idx_hbm.at[pl.ds(base, chunk)], idx_v)
        pltpu.sync_copy(w_hbm.at[pl.ds(base, chunk)], w_v)
        pltpu.sync_copy(zero_hbm, acc_v)

        def body(i, carry):
            off = pl.multiple_of(i * _KNN, 16)
            iv = idx_v[pl.ds(off, _KNN)]
            wv = w_v[pl.ds(off, _KNN)]
            plsc.addupdate_scatter(acc_v, [iv], wv)
            return carry

        jax.lax.fori_loop(0, chunk // _KNN, body, 0)
        pltpu.sync_copy(acc_v, out_hbm.at[wid])

    return sc_scatter


def _matvec_kernel(w_ref, v_ref, o_ref):
    @pl.when(pl.program_id(0) == 0)
    def _init():
        o_ref[...] = jnp.zeros_like(o_ref)

    wsum = jnp.sum(w_ref[...], axis=0)
    prod = v_ref[...] * wsum[:, :, None]
    o_ref[...] += jnp.sum(prod, axis=(0, 1)).reshape(1, _HID)


def _run_matvec(w32, values3, rb):
    nblk = values3.shape[0] // rb
    return pl.pallas_call(
        _matvec_kernel,
        grid=(nblk,),
        in_specs=[
            pl.BlockSpec((_SC_WORKERS, rb, 128), lambda i: (0, i, 0)),
            pl.BlockSpec((rb, 128, _HID), lambda i: (i, 0, 0)),
        ],
        out_specs=pl.BlockSpec((1, _HID), lambda i: (0, 0)),
        out_shape=jax.ShapeDtypeStruct((1, _HID), jnp.float32),
        interpret=_INTERPRET,
    )(w32, values3)


def kernel(x, W1, b1, W2, b2, bn_w, bn_b, keys, values):
    Bx, Sx, H = x.shape
    x2 = x.reshape(Bx * Sx, H)
    n = x2.shape[0]

    x_perm = jnp.transpose(x2.reshape(n // _HEADS, _HEADS, H), (1, 0, 2))
    x_perm = x_perm.reshape(n, H)

    q, psum, psumsq = _run_mlp(x_perm, W1, b1, W2, b2, tb=256)

    tot = n * _HEADS
    csum = psum.reshape(-1, 2 * _HID).sum(axis=0).reshape(_HEADS, _KD).sum(axis=0)
    csumsq = psumsq.reshape(-1, 2 * _HID).sum(axis=0).reshape(_HEADS, _KD).sum(axis=0)
    mean = csum / tot
    var = csumsq / tot - mean * mean
    scale = bn_w / jnp.sqrt(var + 1e-5)
    shift = bn_b - mean * scale
    scale_full = jnp.tile(scale, _HEADS).reshape(1, 2 * _HID)
    shift_full = jnp.tile(shift, _HEADS).reshape(1, 2 * _HID)

    k1t = jnp.transpose(keys[:, 0], (0, 2, 1))
    k2t = jnp.transpose(keys[:, 1], (0, 2, 1))

    tb2 = min(256, n // _HEADS)
    wts, cidx = _run_score(q, scale_full, shift_full, k1t, k2t, tb=tb2)

    npairs = wts.size
    zero = jnp.zeros((_NROWS,), jnp.float32)
    w32 = _make_sc_scatter(npairs)(cidx.reshape(-1), wts.reshape(-1), zero)

    out = _run_matvec(w32.reshape(_SC_WORKERS, 512, 128),
                      values.reshape(512, 128, _HID), rb=16)
    return out.reshape(Bx, _HID)


# speedup vs baseline: 9.9505x; 1.1729x over previous
import functools

import jax
import jax.numpy as jnp
from jax.experimental import pallas as pl
from jax.experimental.pallas import tpu as pltpu
from jax.experimental.pallas import tpu_sc as plsc

_HID = 1024
_KD = 512
_HALF = 256
_HEADS = 4
_SUB = 256
_KNN = 16
_NROWS = _SUB * _SUB

_SC_CORES = 2
_SC_SUBCORES = 16
_SC_WORKERS = _SC_CORES * _SC_SUBCORES

_INTERPRET = False


def _mlp_kernel(x_ref, w1t_ref, b1_ref, w2t_ref, b2_ref, q_ref, s_ref, ss_ref):
    h = jnp.maximum(
        jnp.dot(x_ref[...], w1t_ref[...], preferred_element_type=jnp.float32)
        + b1_ref[...], 0.0)
    q = jnp.dot(h, w2t_ref[...], preferred_element_type=jnp.float32) + b2_ref[...]
    q_ref[...] = q
    s_ref[0] = jnp.sum(q, axis=0, keepdims=True)
    ss_ref[0] = jnp.sum(q * q, axis=0, keepdims=True)


def _run_mlp(x2, W1, b1, W2, b2, tb):
    n = x2.shape[0]
    grid = n // tb
    return pl.pallas_call(
        _mlp_kernel,
        grid=(grid,),
        in_specs=[
            pl.BlockSpec((tb, _HID), lambda i: (i, 0)),
            pl.BlockSpec((_HID, _KD), lambda i: (0, 0)),
            pl.BlockSpec((1, _KD), lambda i: (0, 0)),
            pl.BlockSpec((_KD, 2 * _HID), lambda i: (0, 0)),
            pl.BlockSpec((1, 2 * _HID), lambda i: (0, 0)),
        ],
        out_specs=[
            pl.BlockSpec((tb, 2 * _HID), lambda i: (i, 0)),
            pl.BlockSpec((1, 1, 2 * _HID), lambda i: (i, 0, 0)),
            pl.BlockSpec((1, 1, 2 * _HID), lambda i: (i, 0, 0)),
        ],
        out_shape=[
            jax.ShapeDtypeStruct((n, 2 * _HID), jnp.float32),
            jax.ShapeDtypeStruct((grid, 1, 2 * _HID), jnp.float32),
            jax.ShapeDtypeStruct((grid, 1, 2 * _HID), jnp.float32),
        ],
        interpret=_INTERPRET,
    )(x2, W1.T, b1.reshape(1, _KD), W2.T, b2.reshape(1, 2 * _HID))


def _topk16(s):
    b = jax.lax.bitcast_convert_type(s, jnp.int32)
    lane = jax.lax.broadcasted_iota(jnp.int32, s.shape, 1)
    code = jnp.where(b < 0, lane, jnp.int32(255) - lane)
    cur = jax.lax.bitcast_convert_type((b & jnp.int32(~0xFF)) | code,
                                       jnp.float32)
    ms = []
    for _ in range(_KNN):
        m = jnp.max(cur, axis=1, keepdims=True)
        ms.append(m)
        cur = jnp.where(cur == m, -jnp.inf, cur)
    mb = jax.lax.bitcast_convert_type(jnp.concatenate(ms, axis=1), jnp.int32)
    codek = mb & jnp.int32(0xFF)
    idx = jnp.where(mb < 0, codek, jnp.int32(255) - codek)
    vals = jax.lax.bitcast_convert_type(mb & jnp.int32(~0xFF), jnp.float32)
    return vals, idx


def _sel16(sel, payload):
    out = jnp.zeros_like(payload)
    for c in range(_KNN):
        out = out + jnp.where(sel == c, payload[:, c:c + 1], 0.0)
    return out


_CHUNK = 64


def _score_kernel(q_ref, sc_ref, sh_ref, k1t_ref, k2t_ref, w_ref, ci_ref):
    qn = q_ref[...] * sc_ref[...] + sh_ref[...]
    s1 = jnp.dot(qn[:, :_HALF], k1t_ref[0], preferred_element_type=jnp.float32)
    s2 = jnp.dot(qn[:, _HALF:], k2t_ref[0], preferred_element_type=jnp.float32)
    lane = jax.lax.broadcasted_iota(jnp.int32, (_KNN, _KNN * _KNN), 1)
    row = jax.lax.broadcasted_iota(jnp.int32, (_KNN, _KNN * _KNN), 0)
    selA = (lane // _KNN == row).astype(jnp.float32)
    selB = (lane % _KNN == row).astype(jnp.float32)
    hp = jax.lax.Precision.HIGHEST
    rows = s1.shape[0]
    ws, cis = [], []
    for r0 in range(0, rows, _CHUNK):
        v1, i1 = _topk16(s1[r0:r0 + _CHUNK])
        v2, i2 = _topk16(s2[r0:r0 + _CHUNK])
        all_s = (jnp.dot(v1, selA, preferred_element_type=jnp.float32,
                         precision=hp)
                 + jnp.dot(v2, selB, preferred_element_type=jnp.float32,
                           precision=hp))
        vt, lt = _topk16(all_s)
        at = _sel16(lt // _KNN, i1.astype(jnp.float32))
        bt = _sel16(lt % _KNN, i2.astype(jnp.float32))
        m = jnp.max(vt, axis=1, keepdims=True)
        e = jnp.exp(vt - m)
        ws.append(e / jnp.sum(e, axis=1, keepdims=True))
        cis.append((at * _SUB + bt).astype(jnp.int32))
    w_ref[0] = jnp.concatenate(ws, axis=0)
    ci_ref[0] = jnp.concatenate(cis, axis=0)


def _run_score(q_bykey, scale_full, shift_full, k1t, k2t, tb):
    n = q_bykey.shape[0]
    per_key = n // _HEADS
    grid_g = per_key // tb
    ng = _HEADS * _HEADS
    return pl.pallas_call(
        _score_kernel,
        grid=(_HEADS, _HEADS, grid_g),
        in_specs=[
            pl.BlockSpec((tb, _KD), lambda k, h, g: (k * (per_key // tb) + g, h)),
            pl.BlockSpec((1, _KD), lambda k, h, g: (0, h)),
            pl.BlockSpec((1, _KD), lambda k, h, g: (0, h)),
            pl.BlockSpec((1, _HALF, _SUB), lambda k, h, g: (k, 0, 0)),
            pl.BlockSpec((1, _HALF, _SUB), lambda k, h, g: (k, 0, 0)),
        ],
        out_specs=[
            pl.BlockSpec((1, tb, _KNN), lambda k, h, g: (k * _HEADS + h, g, 0)),
            pl.BlockSpec((1, tb, _KNN), lambda k, h, g: (k * _HEADS + h, g, 0)),
        ],
        out_shape=[
            jax.ShapeDtypeStruct((ng, per_key, _KNN), jnp.float32),
            jax.ShapeDtypeStruct((ng, per_key, _KNN), jnp.int32),
        ],
        interpret=_INTERPRET,
    )(q_bykey, scale_full, shift_full, k1t, k2t)


def _make_sc_scatter(npairs):
    chunk = npairs // _SC_WORKERS
    mesh = plsc.VectorSubcoreMesh(core_axis_name="c", subcore_axis_name="s",
                                  num_cores=_SC_CORES,
                                  num_subcores=_SC_SUBCORES)

    @functools.partial(
        pl.kernel, mesh=mesh,
        compiler_params=pltpu.CompilerParams(needs_layout_passes=False),
        out_type=jax.ShapeDtypeStruct((_SC_WORKERS, _NROWS), jnp.float32),
        scratch_types=[
            pltpu.VMEM((chunk,), jnp.int32),
            pltpu.VMEM((chunk,), jnp.float32),
            pltpu.VMEM((_NROWS,), jnp.float32),
        ],
    )
    def sc_scatter(idx_hbm, w_hbm, zero_hbm, out_hbm, idx_v, w_v, acc_v):
        wid = (jax.lax.axis_index("s") * _SC_CORES + jax.lax.axis_index("c"))
        base = wid * chunk
        pltpu.sync_copy(idx_hbm.at[pl.ds(base, chunk)], idx_v)
        pltpu.sync_copy(w_hbm.at[pl.ds(base, chunk)], w_v)
        pltpu.sync_copy(zero_hbm, acc_v)

        def body(i, carry):
            off = pl.multiple_of(i * _KNN, 16)
            iv = idx_v[pl.ds(off, _KNN)]
            wv = w_v[pl.ds(off, _KNN)]
            plsc.addupdate_scatter(acc_v, [iv], wv)
            return carry

        jax.lax.fori_loop(0, chunk // _KNN, body, 0)
        pltpu.sync_copy(acc_v, out_hbm.at[wid])

    return sc_scatter


def _matvec_kernel(w_ref, v_ref, o_ref):
    @pl.when(pl.program_id(0) == 0)
    def _init():
        o_ref[...] = jnp.zeros_like(o_ref)

    wsum = jnp.sum(w_ref[...], axis=0)
    prod = v_ref[...] * wsum[:, :, None]
    o_ref[...] += jnp.sum(prod, axis=(0, 1)).reshape(1, _HID)


def _run_matvec(w32, values3, rb):
    nblk = values3.shape[0] // rb
    return pl.pallas_call(
        _matvec_kernel,
        grid=(nblk,),
        in_specs=[
            pl.BlockSpec((_SC_WORKERS, rb, 128), lambda i: (0, i, 0)),
            pl.BlockSpec((rb, 128, _HID), lambda i: (i, 0, 0)),
        ],
        out_specs=pl.BlockSpec((1, _HID), lambda i: (0, 0)),
        out_shape=jax.ShapeDtypeStruct((1, _HID), jnp.float32),
        interpret=_INTERPRET,
    )(w32, values3)


def kernel(x, W1, b1, W2, b2, bn_w, bn_b, keys, values):
    Bx, Sx, H = x.shape
    x2 = x.reshape(Bx * Sx, H)
    n = x2.shape[0]

    x_perm = jnp.transpose(x2.reshape(n // _HEADS, _HEADS, H), (1, 0, 2))
    x_perm = x_perm.reshape(n, H)

    q, psum, psumsq = _run_mlp(x_perm, W1, b1, W2, b2, tb=256)

    tot = n * _HEADS
    csum = psum.reshape(-1, 2 * _HID).sum(axis=0).reshape(_HEADS, _KD).sum(axis=0)
    csumsq = psumsq.reshape(-1, 2 * _HID).sum(axis=0).reshape(_HEADS, _KD).sum(axis=0)
    mean = csum / tot
    var = csumsq / tot - mean * mean
    scale = bn_w / jnp.sqrt(var + 1e-5)
    shift = bn_b - mean * scale
    scale_full = jnp.tile(scale, _HEADS).reshape(1, 2 * _HID)
    shift_full = jnp.tile(shift, _HEADS).reshape(1, 2 * _HID)

    k1t = jnp.transpose(keys[:, 0], (0, 2, 1))
    k2t = jnp.transpose(keys[:, 1], (0, 2, 1))

    tb2 = min(256, n // _HEADS)
    wts, cidx = _run_score(q, scale_full, shift_full, k1t, k2t, tb=tb2)

    npairs = wts.size
    zero = jnp.zeros((_NROWS,), jnp.float32)
    w32 = _make_sc_scatter(npairs)(cidx.reshape(-1), wts.reshape(-1), zero)

    out = _run_matvec(w32.reshape(_SC_WORKERS, 512, 128),
                      values.reshape(512, 128, _HID), rb=16)
    return out.reshape(Bx, _HID)
